# baseline (speedup 1.0000x reference)
import functools

import jax
import jax.numpy as jnp
from jax.experimental import pallas as pl
from jax.experimental.pallas import tpu as pltpu
from jax.experimental.pallas import tpu_sc as plsc

D = 768
S = 8
H = 12
E = 16
TOPK = 2
TOPK_READ = 4
TOPK_WRITE = 4
HD = D // H
SCALE = 1.0 / (HD ** 0.5)
EPS = 1e-5
NEG = -1e30

F32 = jnp.float32
BF16 = jnp.bfloat16

TT = 512
TQ = 512


def _mm(a, b):
    return jnp.dot(a, b, preferred_element_type=F32)


def _mml(a, b):
    return jnp.dot(a.astype(BF16), b.astype(BF16), preferred_element_type=F32)


def _mmtl(a, b):
    return jax.lax.dot_general(
        a.astype(BF16), b.astype(BF16), (((1,), (1,)), ((), ())),
        preferred_element_type=F32)


def _outer(a, b):
    return jax.lax.dot_general(
        a, b, (((0,), (0,)), ((), ())), preferred_element_type=F32)


def _ln(x, g, b):
    m = jnp.mean(x, axis=-1, keepdims=True)
    v = jnp.mean((x - m) ** 2, axis=-1, keepdims=True)
    return (x - m) * jax.lax.rsqrt(v + EPS) * g + b


def _gelu(x):
    return 0.5 * x * (1.0 + jax.lax.erf(x * (2.0 ** -0.5)))


def _topk_dense(scores, k):
    n = scores.shape[-1]
    iota = jax.lax.broadcasted_iota(jnp.int32, (1, n), 1).astype(F32)
    sel = scores
    ohs = []
    for _ in range(k):
        m = jnp.max(sel, axis=-1, keepdims=True)
        cand = jnp.where(sel >= m, iota, F32(n))
        imin = jnp.min(cand, axis=-1, keepdims=True)
        oh = (iota == imin).astype(F32)
        ohs.append(oh)
        sel = jnp.where(oh > 0, F32(NEG), sel)
    mask = ohs[0]
    for oh in ohs[1:]:
        mask = mask + oh
    wf = jax.nn.softmax(scores, axis=-1)
    wm = wf * mask
    wm = wm / (jnp.sum(wm, axis=-1, keepdims=True) + 1e-9)
    return ohs, wm, mask, iota


def _se_proj_body(x_ref, pe_ref, sew_ref, seb_ref, g_ref, b_ref,
                  wk_ref, wv_ref, bk_ref, bv_ref, k1_ref, v1_ref):
    xb = x_ref[0]
    lsx = _mmtl(xb, sew_ref[...]) + seb_ref[...] + pe_ref[...]
    ln = _ln(lsx, g_ref[...], b_ref[...])
    k1_ref[0] = (_mmtl(ln, wk_ref[0]) + bk_ref[...]).astype(BF16)
    v1_ref[0] = (_mmtl(ln, wv_ref[0]) + bv_ref[...]).astype(BF16)


def _oe_proj_body(x_ref, pe_ref, oew_ref, oeb_ref, qg_ref, qb_ref,
                  kg_ref, kb_ref, wqkv_ref, bq_ref, bk_ref, bv_ref,
                  h0_ref, q2_ref, k2_ref, v2_ref):
    xb = x_ref[0]
    h0 = _mmtl(xb, oew_ref[...]) + oeb_ref[...] + pe_ref[...]
    h0_ref[0] = h0.astype(BF16)
    lq = _ln(h0, qg_ref[...], qb_ref[...])
    q2_ref[0] = ((_mmtl(lq, wqkv_ref[0]) + bq_ref[...]) * SCALE).astype(BF16)
    lk = _ln(h0, kg_ref[...], kb_ref[...])
    k2_ref[0] = (_mmtl(lk, wqkv_ref[1]) + bk_ref[...]).astype(BF16)
    v2_ref[0] = (_mmtl(lk, wqkv_ref[2]) + bv_ref[...]).astype(BF16)


def _read_attn_body(st_ref, k1_ref, v1_ref, rw_ref, qg_ref, qb_ref,
                    kvg_ref, kvb_ref, wqkv_ref, bq_ref, bk_ref, bv_ref,
                    ow_ref, ob_ref, lr1_ref):
    latent = st_ref[0]
    scores = jnp.sum(latent * rw_ref[...], axis=-1, keepdims=True)
    scores = scores.reshape(1, S)
    ohs, wm, _, _ = _topk_dense(scores, TOPK_READ)
    rows = jnp.concatenate(ohs, axis=0)
    rw = jnp.concatenate(
        [jnp.sum(wm * oh, axis=-1, keepdims=True) for oh in ohs], axis=0)
    lr0 = _mm(rows * rw, latent)

    q = _ln(lr0, qg_ref[...], qb_ref[...])
    qp = (_mmtl(q, wqkv_ref[0]) + bq_ref[...]) * SCALE
    kv0 = _ln(lr0, kvg_ref[...], kvb_ref[...])
    k0 = _mmtl(kv0, wqkv_ref[1]) + bk_ref[...]
    v0 = _mmtl(kv0, wqkv_ref[2]) + bv_ref[...]
    k1 = k1_ref[0]
    v1 = v1_ref[0]

    heads = []
    for h in range(H):
        sl = slice(h * HD, (h + 1) * HD)
        qh = qp[:, sl]
        s0 = _mmtl(qh, k0[:, sl])
        s1 = _mmtl(qh, k1[:, sl])
        m = jnp.maximum(jnp.max(s0, axis=-1, keepdims=True),
                        jnp.max(s1, axis=-1, keepdims=True))
        p0 = jnp.exp(s0 - m)
        p1 = jnp.exp(s1 - m)
        l = (jnp.sum(p0, axis=-1, keepdims=True)
             + jnp.sum(p1, axis=-1, keepdims=True))
        heads.append((_mml(p0, v0[:, sl]) + _mml(p1, v1[:, sl])) / l)
    att = jnp.concatenate(heads, axis=1)
    lr1_ref[0] = lr0 + _mmtl(att, ow_ref[...]) + ob_ref[...]


def _gate_body(lr1_ref, st_ref, fg_ref, fb_ref, w1_ref, b1_ref,
               w2_ref, b2_ref, mg_ref, mb_ref, gw_ref, sg_ref, sb_ref,
               ww_ref, lr2_ref, eidx_ref, ew_ref, ohe_ref, wmw_ref, wmask_ref):
    lr1 = lr1_ref[0]
    t = _ln(lr1, fg_ref[...], fb_ref[...])
    t = _gelu(_mmtl(t, w1_ref[...]) + b1_ref[...])
    lr2 = lr1 + _mmtl(t, w2_ref[...]) + b2_ref[...]
    lr2_ref[0] = lr2

    pooled = _ln(jnp.mean(lr2, axis=0, keepdims=True), mg_ref[...], mb_ref[...])
    logits = _mmtl(pooled, gw_ref[...])
    ohs, wm, _, iota = _topk_dense(logits, TOPK)
    eidx = jnp.concatenate(
        [jnp.sum(iota * oh, axis=-1, keepdims=True) for oh in ohs], axis=1)
    ew = jnp.concatenate(
        [jnp.sum(wm * oh, axis=-1, keepdims=True) for oh in ohs], axis=1)
    eidx_ref[0] = eidx.astype(jnp.int32)
    ew_ref[0] = ew
    ohe_ref[0] = jnp.concatenate(ohs, axis=0)

    latent = st_ref[0]
    wsc = jnp.sum(_ln(latent, sg_ref[...], sb_ref[...]) * ww_ref[...],
                  axis=-1, keepdims=True).reshape(1, S)
    _, wmw, wmask, _ = _topk_dense(wsc, TOPK_WRITE)
    wmw_ref[0] = wmw
    wmask_ref[0] = wmask


GWIN = 128


def _gather_experts(exp_w, eidx_flat):
    n = eidx_flat.shape[0]
    rpe = D * D // 128
    rows = (eidx_flat[:, None] * rpe
            + jnp.arange(rpe, dtype=jnp.int32)[None, :]).reshape(1, n * rpe)
    w2d = exp_w.reshape(E * rpe, 128)
    mesh = plsc.VectorSubcoreMesh(core_axis_name="c", subcore_axis_name="s")

    @functools.partial(
        pl.kernel,
        out_type=jax.ShapeDtypeStruct((n * rpe, 128), F32),
        mesh=mesh,
    )
    def gather_kernel(w_hbm, i_hbm, o_hbm):
        def body(i_vmem, o_vmem):
            pltpu.sync_copy(w_hbm.at[i_vmem.at[0]], o_vmem)

        pltpu.emit_pipeline(
            body,
            grid=(n * rpe // GWIN,),
            in_specs=[pl.BlockSpec((1, GWIN), lambda i: (0, i))],
            out_specs=[pl.BlockSpec((GWIN, 128), lambda i: (i, 0))],
            core_axis_name=("c", "s"),
            dimension_semantics=(pltpu.PARALLEL,),
        )(i_hbm, o_hbm)

    return gather_kernel(w2d, rows).reshape(n, D, D)


def _moe_state_body(lr2_ref, wsel_ref, expb_ref, ohe_ref, ew_ref, st_ref,
                    wmw_ref, wmask_ref, lg_ref, lb_ref, okg_ref, okb_ref,
                    wk_ref, wv_ref, bk_ref, bv_ref,
                    stn_ref, ks_ref, vs_ref):
    lr = lr2_ref[0]
    acc = jnp.zeros_like(lr)
    bsel = _mm(ohe_ref[0], expb_ref[...])
    for k in range(TOPK):
        w = wsel_ref[k]
        yk = jax.nn.relu(_mml(lr, w) + bsel[k:k + 1, :])
        acc = acc + ew_ref[0, :, k:k + 1] * yk
    lr3 = lr + acc

    meanv = jnp.mean(lr3, axis=0, keepdims=True)
    latent = st_ref[0]
    st_upd = latent + _outer(wmw_ref[0], meanv)
    st_norm = _ln(st_upd, lg_ref[...], lb_ref[...])
    blend = 0.5 * latent + 0.5 * st_norm
    maskm = _outer(wmask_ref[0], jnp.ones((1, D), F32))
    st_new = jnp.where(maskm > 0.5, blend, latent)
    stn_ref[0] = st_new

    kvn = _ln(st_new, okg_ref[...], okb_ref[...])
    ks_ref[0] = (_mmtl(kvn, wk_ref[0]) + bk_ref[...]).astype(BF16)
    vs_ref[0] = (_mmtl(kvn, wv_ref[0]) + bv_ref[...]).astype(BF16)


def _out_attn_body(q_ref, k2_ref, v2_ref, ks_ref, vs_ref, ao_ref):
    q = q_ref[0]
    k2 = k2_ref[0]
    v2 = v2_ref[0]
    ks = ks_ref[0]
    vs = vs_ref[0]
    for h in range(H):
        sl = slice(h * HD, (h + 1) * HD)
        qh = q[:, sl]
        s0 = _mmtl(qh, ks[:, sl])
        s1 = _mmtl(qh, k2[:, sl])
        m = jnp.maximum(jnp.max(s0, axis=-1, keepdims=True),
                        jnp.max(s1, axis=-1, keepdims=True))
        p0 = jnp.exp(s0 - m)
        p1 = jnp.exp(s1 - m)
        l = (jnp.sum(p0, axis=-1, keepdims=True)
             + jnp.sum(p1, axis=-1, keepdims=True))
        ao_ref[0, :, sl] = ((_mml(p0, vs[:, sl]) + _mml(p1, v2[:, sl]))
                            / l).astype(BF16)


def _final_body(ao_ref, h0_ref, ow_ref, ob_ref, fg_ref, fb_ref,
                w1_ref, b1_ref, w2_ref, b2_ref, opw_ref, opb_ref, y_ref):
    h1 = h0_ref[0].astype(F32) + _mmtl(ao_ref[0], ow_ref[...]) + ob_ref[...]
    t = _ln(h1, fg_ref[...], fb_ref[...])
    t = _gelu(_mmtl(t, w1_ref[...]) + b1_ref[...])
    h2 = h1 + _mmtl(t, w2_ref[...]) + b2_ref[...]
    y_ref[0] = _mmtl(h2, opw_ref[...]) + opb_ref[...]


def _full(shape=(D, D)):
    return pl.BlockSpec(shape, lambda *_: tuple(0 for _ in shape))


def kernel(x, state_flat, params):
    p = params
    B, T, _ = x.shape
    state = state_flat.reshape(B, S, D)

    pos = jnp.arange(T, dtype=F32)[:, None]
    f = float(S) ** (jnp.arange(D // 2).astype(F32) / (D // 2))
    pe = jnp.concatenate([jnp.sin(pos / f), jnp.cos(pos / f)], axis=-1)

    r2 = lambda v: v.reshape(1, D)
    smha_w3 = p['smha_w'].reshape(3, D, D)
    omha_w3 = p['omha_w'].reshape(3, D, D)
    bq1, bk1, bv1 = (p['smha_b'][:D][None], p['smha_b'][D:2 * D][None],
                     p['smha_b'][2 * D:][None])
    bq2, bk2, bv2 = (p['omha_b'][:D][None], p['omha_b'][D:2 * D][None],
                     p['omha_b'][2 * D:][None])

    vec = _full((1, D))
    w3 = lambda k, n=1: pl.BlockSpec((n, D, D), lambda *_: (k, 0, 0))
    tile = lambda: pl.BlockSpec((1, TT, D), lambda b, t: (b, t, 0))
    fseq = lambda n: pl.BlockSpec((1, n, D), lambda b: (b, 0, 0))

    k1, v1 = pl.pallas_call(
        _se_proj_body,
        grid=(B, T // TT),
        in_specs=[tile(), pl.BlockSpec((TT, D), lambda b, t: (t, 0)),
                  _full(), vec, vec, vec, w3(1), w3(2), vec, vec],
        out_specs=[tile(), tile()],
        out_shape=[jax.ShapeDtypeStruct((B, T, D), BF16)] * 2,
    )(x, pe, p['se_W'], r2(p['se_b']), r2(p['sln_kv_g']), r2(p['sln_kv_b']),
      smha_w3, smha_w3, bk1, bv1)

    h0, q2, k2, v2 = pl.pallas_call(
        _oe_proj_body,
        grid=(B, T // TT),
        in_specs=[tile(), pl.BlockSpec((TT, D), lambda b, t: (t, 0)),
                  _full(), vec, vec, vec, vec, vec,
                  w3(0, 3), vec, vec, vec],
        out_specs=[tile(), tile(), tile(), tile()],
        out_shape=[jax.ShapeDtypeStruct((B, T, D), BF16)] * 4,
    )(x, pe, p['oe_W'], r2(p['oe_b']), r2(p['oln_q_g']), r2(p['oln_q_b']),
      r2(p['oln_kv_g']), r2(p['oln_kv_b']), omha_w3, bq2, bk2, bv2)

    lr1 = pl.pallas_call(
        _read_attn_body,
        grid=(B,),
        in_specs=[fseq(S), fseq(T), fseq(T), vec, vec, vec, vec, vec,
                  w3(0, 3), vec, vec, vec, _full(), vec],
        out_specs=[fseq(TOPK_READ)],
        out_shape=[jax.ShapeDtypeStruct((B, TOPK_READ, D), F32)],
    )(state, k1, v1, p['read_w'].reshape(1, D),
      r2(p['sln_q_g']), r2(p['sln_q_b']),
      r2(p['sln_kv_g']), r2(p['sln_kv_b']),
      smha_w3, bq1, bk1, bv1, p['smha_ow'], r2(p['smha_ob']))[0]

    sm = lambda n: pl.BlockSpec((1, 1, n), lambda b: (b, 0, 0))
    lr2, eidx, ew, ohe, wmw, wmask = pl.pallas_call(
        _gate_body,
        grid=(B,),
        in_specs=[fseq(TOPK_READ), fseq(S), vec, vec, _full(), vec,
                  _full(), vec, vec, vec, _full((E, D)), vec, vec, vec],
        out_specs=[fseq(TOPK_READ), sm(TOPK), sm(TOPK),
                   pl.BlockSpec((1, TOPK, E), lambda b: (b, 0, 0)),
                   sm(S), sm(S)],
        out_shape=[jax.ShapeDtypeStruct((B, TOPK_READ, D), F32),
                   jax.ShapeDtypeStruct((B, 1, TOPK), jnp.int32),
                   jax.ShapeDtypeStruct((B, 1, TOPK), F32),
                   jax.ShapeDtypeStruct((B, TOPK, E), F32),
                   jax.ShapeDtypeStruct((B, 1, S), F32),
                   jax.ShapeDtypeStruct((B, 1, S), F32)],
    )(lr1, state, r2(p['sln_ffn_g']), r2(p['sln_ffn_b']),
      p['sffn_w1'], r2(p['sffn_b1']), p['sffn_w2'], r2(p['sffn_b2']),
      r2(p['sln_moe_g']), r2(p['sln_moe_b']), p['gate_W'],
      r2(p['sln_slot_g']), r2(p['sln_slot_b']), p['write_w'].reshape(1, D))

    wsel = _gather_experts(p['exp_W'], eidx.reshape(-1))

    st_new, ks, vs = pl.pallas_call(
        _moe_state_body,
        grid=(B,),
        in_specs=[fseq(TOPK_READ),
                  pl.BlockSpec((TOPK, D, D), lambda b: (b, 0, 0)),
                  _full((E, D)),
                  pl.BlockSpec((1, TOPK, E), lambda b: (b, 0, 0)),
                  sm(TOPK), fseq(S), sm(S), sm(S),
                  vec, vec, vec, vec, w3(1), w3(2), vec, vec],
        out_specs=[fseq(S), fseq(S), fseq(S)],
        out_shape=[jax.ShapeDtypeStruct((B, S, D), F32),
                   jax.ShapeDtypeStruct((B, S, D), BF16),
                   jax.ShapeDtypeStruct((B, S, D), BF16)],
    )(lr2, wsel, p['exp_b'], ohe, ew, state, wmw, wmask,
      r2(p['ln_state_g']), r2(p['ln_state_b']),
      r2(p['oln_kv_g']), r2(p['oln_kv_b']), omha_w3, omha_w3, bk2, bv2)

    ao = pl.pallas_call(
        _out_attn_body,
        grid=(B, T // TQ),
        in_specs=[pl.BlockSpec((1, TQ, D), lambda b, t: (b, t, 0)),
                  pl.BlockSpec((1, T, D), lambda b, t: (b, 0, 0)),
                  pl.BlockSpec((1, T, D), lambda b, t: (b, 0, 0)),
                  pl.BlockSpec((1, S, D), lambda b, t: (b, 0, 0)),
                  pl.BlockSpec((1, S, D), lambda b, t: (b, 0, 0))],
        out_specs=[pl.BlockSpec((1, TQ, D), lambda b, t: (b, t, 0))],
        out_shape=[jax.ShapeDtypeStruct((B, T, D), BF16)],
    )(q2, k2, v2, ks, vs)[0]

    y = pl.pallas_call(
        _final_body,
        grid=(B, T // TT),
        in_specs=[tile(), tile(), _full(), vec, vec, vec,
                  _full(), vec, _full(), vec, _full(), vec],
        out_specs=[tile()],
        out_shape=[jax.ShapeDtypeStruct((B, T, D), F32)],
    )(ao, h0, p['omha_ow'], r2(p['omha_ob']),
      r2(p['oln_ffn_g']), r2(p['oln_ffn_b']),
      p['offn_w1'], r2(p['offn_b1']), p['offn_w2'], r2(p['offn_b2']),
      p['op_W'], r2(p['op_b']))

    return y[0], st_new.reshape(B, S * D)


# speedup vs baseline: 1.0086x; 1.0086x over previous
import functools

import jax
import jax.numpy as jnp
from jax.experimental import pallas as pl
from jax.experimental.pallas import tpu as pltpu
from jax.experimental.pallas import tpu_sc as plsc

D = 768
S = 8
H = 12
E = 16
TOPK = 2
TOPK_READ = 4
TOPK_WRITE = 4
HD = D // H
SCALE = 1.0 / (HD ** 0.5)
EPS = 1e-5
NEG = -1e30

F32 = jnp.float32
BF16 = jnp.bfloat16

TT = 512
TQ = 512


def _mm(a, b):
    return jnp.dot(a, b, preferred_element_type=F32)


def _mml(a, b):
    return jnp.dot(a.astype(BF16), b.astype(BF16), preferred_element_type=F32)


def _mmtl(a, b):
    return jax.lax.dot_general(
        a.astype(BF16), b.astype(BF16), (((1,), (1,)), ((), ())),
        preferred_element_type=F32)


def _outer(a, b):
    return jax.lax.dot_general(
        a, b, (((0,), (0,)), ((), ())), preferred_element_type=F32)


def _ln(x, g, b):
    m = jnp.mean(x, axis=-1, keepdims=True)
    v = jnp.mean((x - m) ** 2, axis=-1, keepdims=True)
    return (x - m) * jax.lax.rsqrt(v + EPS) * g + b


def _gelu(x):
    return 0.5 * x * (1.0 + jax.lax.erf(x * (2.0 ** -0.5)))


def _topk_dense(scores, k):
    n = scores.shape[-1]
    iota = jax.lax.broadcasted_iota(jnp.int32, (1, n), 1).astype(F32)
    sel = scores
    ohs = []
    for _ in range(k):
        m = jnp.max(sel, axis=-1, keepdims=True)
        cand = jnp.where(sel >= m, iota, F32(n))
        imin = jnp.min(cand, axis=-1, keepdims=True)
        oh = (iota == imin).astype(F32)
        ohs.append(oh)
        sel = jnp.where(oh > 0, F32(NEG), sel)
    mask = ohs[0]
    for oh in ohs[1:]:
        mask = mask + oh
    wf = jax.nn.softmax(scores, axis=-1)
    wm = wf * mask
    wm = wm / (jnp.sum(wm, axis=-1, keepdims=True) + 1e-9)
    return ohs, wm, mask, iota


def _se_proj_body(x_ref, pe_ref, sew_ref, seb_ref, g_ref, b_ref,
                  wk_ref, wv_ref, bk_ref, bv_ref, k1_ref, v1_ref):
    xb = x_ref[0]
    lsx = _mmtl(xb, sew_ref[...]) + seb_ref[...] + pe_ref[...]
    ln = _ln(lsx, g_ref[...], b_ref[...])
    k1_ref[0] = (_mmtl(ln, wk_ref[0]) + bk_ref[...]).astype(BF16)
    v1_ref[0] = (_mmtl(ln, wv_ref[0]) + bv_ref[...]).astype(BF16)


def _oe_proj_body(x_ref, pe_ref, oew_ref, oeb_ref, qg_ref, qb_ref,
                  kg_ref, kb_ref, wqkv_ref, bq_ref, bk_ref, bv_ref,
                  h0_ref, q2_ref, k2_ref, v2_ref):
    xb = x_ref[0]
    h0 = _mmtl(xb, oew_ref[...]) + oeb_ref[...] + pe_ref[...]
    h0_ref[0] = h0.astype(BF16)
    lq = _ln(h0, qg_ref[...], qb_ref[...])
    q2_ref[0] = ((_mmtl(lq, wqkv_ref[0]) + bq_ref[...]) * SCALE).astype(BF16)
    lk = _ln(h0, kg_ref[...], kb_ref[...])
    k2_ref[0] = (_mmtl(lk, wqkv_ref[1]) + bk_ref[...]).astype(BF16)
    v2_ref[0] = (_mmtl(lk, wqkv_ref[2]) + bv_ref[...]).astype(BF16)


def _read_attn_body(st_ref, k1_ref, v1_ref, rw_ref, qg_ref, qb_ref,
                    kvg_ref, kvb_ref, wqkv_ref, bq_ref, bk_ref, bv_ref,
                    ow_ref, ob_ref, lr1_ref):
    latent = st_ref[0]
    scores = jnp.sum(latent * rw_ref[...], axis=-1, keepdims=True)
    scores = scores.reshape(1, S)
    ohs, wm, _, _ = _topk_dense(scores, TOPK_READ)
    rows = jnp.concatenate(ohs, axis=0)
    rw = jnp.concatenate(
        [jnp.sum(wm * oh, axis=-1, keepdims=True) for oh in ohs], axis=0)
    lr0 = _mm(rows * rw, latent)

    q = _ln(lr0, qg_ref[...], qb_ref[...])
    qp = (_mmtl(q, wqkv_ref[0]) + bq_ref[...]) * SCALE
    kv0 = _ln(lr0, kvg_ref[...], kvb_ref[...])
    k0 = _mmtl(kv0, wqkv_ref[1]) + bk_ref[...]
    v0 = _mmtl(kv0, wqkv_ref[2]) + bv_ref[...]
    k1 = k1_ref[0]
    v1 = v1_ref[0]

    heads = []
    for h in range(H):
        sl = slice(h * HD, (h + 1) * HD)
        qh = qp[:, sl]
        s0 = _mmtl(qh, k0[:, sl])
        s1 = _mmtl(qh, k1[:, sl])
        m = jnp.maximum(jnp.max(s0, axis=-1, keepdims=True),
                        jnp.max(s1, axis=-1, keepdims=True))
        p0 = jnp.exp(s0 - m)
        p1 = jnp.exp(s1 - m)
        l = (jnp.sum(p0, axis=-1, keepdims=True)
             + jnp.sum(p1, axis=-1, keepdims=True))
        heads.append((_mml(p0, v0[:, sl]) + _mml(p1, v1[:, sl])) / l)
    att = jnp.concatenate(heads, axis=1)
    lr1_ref[0] = lr0 + _mmtl(att, ow_ref[...]) + ob_ref[...]


def _gate_body(lr1_ref, st_ref, fg_ref, fb_ref, w1_ref, b1_ref,
               w2_ref, b2_ref, mg_ref, mb_ref, gw_ref, sg_ref, sb_ref,
               ww_ref, lr2_ref, eidx_ref, ew_ref, ohe_ref, wmw_ref, wmask_ref):
    lr1 = lr1_ref[0]
    t = _ln(lr1, fg_ref[...], fb_ref[...])
    t = _gelu(_mmtl(t, w1_ref[...]) + b1_ref[...])
    lr2 = lr1 + _mmtl(t, w2_ref[...]) + b2_ref[...]
    lr2_ref[0] = lr2

    pooled = _ln(jnp.mean(lr2, axis=0, keepdims=True), mg_ref[...], mb_ref[...])
    logits = _mmtl(pooled, gw_ref[...])
    ohs, wm, _, iota = _topk_dense(logits, TOPK)
    eidx = jnp.concatenate(
        [jnp.sum(iota * oh, axis=-1, keepdims=True) for oh in ohs], axis=1)
    ew = jnp.concatenate(
        [jnp.sum(wm * oh, axis=-1, keepdims=True) for oh in ohs], axis=1)
    eidx_ref[0] = eidx.astype(jnp.int32)
    ew_ref[0] = ew
    ohe_ref[0] = jnp.concatenate(ohs, axis=0)

    latent = st_ref[0]
    wsc = jnp.sum(_ln(latent, sg_ref[...], sb_ref[...]) * ww_ref[...],
                  axis=-1, keepdims=True).reshape(1, S)
    _, wmw, wmask, _ = _topk_dense(wsc, TOPK_WRITE)
    wmw_ref[0] = wmw
    wmask_ref[0] = wmask


GWIN = 128


def _gather_experts(exp_w, eidx_flat):
    n = eidx_flat.shape[0]
    rpe = D * D // 128
    rows = (eidx_flat[:, None] * rpe
            + jnp.arange(rpe, dtype=jnp.int32)[None, :]).reshape(1, n * rpe)
    w2d = exp_w.reshape(E * rpe, 128)
    mesh = plsc.VectorSubcoreMesh(core_axis_name="c", subcore_axis_name="s")

    @functools.partial(
        pl.kernel,
        out_type=jax.ShapeDtypeStruct((n * rpe, 128), F32),
        mesh=mesh,
    )
    def gather_kernel(w_hbm, i_hbm, o_hbm):
        def body(i_vmem, o_vmem):
            pltpu.sync_copy(w_hbm.at[i_vmem.at[0]], o_vmem)

        pltpu.emit_pipeline(
            body,
            grid=(n * rpe // GWIN,),
            in_specs=[pl.BlockSpec((1, GWIN), lambda i: (0, i))],
            out_specs=[pl.BlockSpec((GWIN, 128), lambda i: (i, 0))],
            core_axis_name=("c", "s"),
            dimension_semantics=(pltpu.PARALLEL,),
        )(i_hbm, o_hbm)

    return gather_kernel(w2d, rows).reshape(n, D, D)


def _moe_state_body(lr2_ref, wsel_ref, expb_ref, ohe_ref, ew_ref, st_ref,
                    wmw_ref, wmask_ref, lg_ref, lb_ref, okg_ref, okb_ref,
                    wk_ref, wv_ref, bk_ref, bv_ref,
                    stn_ref, ks_ref, vs_ref):
    lr = lr2_ref[0]
    acc = jnp.zeros_like(lr)
    bsel = _mm(ohe_ref[0], expb_ref[...])
    for k in range(TOPK):
        w = wsel_ref[k]
        yk = jax.nn.relu(_mml(lr, w) + bsel[k:k + 1, :])
        acc = acc + ew_ref[0, :, k:k + 1] * yk
    lr3 = lr + acc

    meanv = jnp.mean(lr3, axis=0, keepdims=True)
    latent = st_ref[0]
    st_upd = latent + _outer(wmw_ref[0], meanv)
    st_norm = _ln(st_upd, lg_ref[...], lb_ref[...])
    blend = 0.5 * latent + 0.5 * st_norm
    maskm = _outer(wmask_ref[0], jnp.ones((1, D), F32))
    st_new = jnp.where(maskm > 0.5, blend, latent)
    stn_ref[0] = st_new

    kvn = _ln(st_new, okg_ref[...], okb_ref[...])
    ks_ref[0] = (_mmtl(kvn, wk_ref[0]) + bk_ref[...]).astype(BF16)
    vs_ref[0] = (_mmtl(kvn, wv_ref[0]) + bv_ref[...]).astype(BF16)


def _out_attn_body(q_ref, k2_ref, v2_ref, ks_ref, vs_ref, ao_ref):
    q = q_ref[0]
    k2 = k2_ref[0]
    v2 = v2_ref[0]
    ks = ks_ref[0]
    vs = vs_ref[0]
    for h in range(H):
        sl = slice(h * HD, (h + 1) * HD)
        qh = q[:, sl]
        s0 = _mmtl(qh, ks[:, sl])
        s1 = _mmtl(qh, k2[:, sl])
        m = jnp.maximum(jnp.max(s0, axis=-1, keepdims=True),
                        jnp.max(s1, axis=-1, keepdims=True))
        p0 = jnp.exp(s0 - m)
        p1 = jnp.exp(s1 - m).astype(BF16)
        l = (jnp.sum(p0, axis=-1, keepdims=True)
             + jnp.sum(p1.astype(F32), axis=-1, keepdims=True))
        ao_ref[0, :, sl] = ((_mml(p0, vs[:, sl]) + _mml(p1, v2[:, sl]))
                            / l).astype(BF16)


def _final_body(ao_ref, h0_ref, ow_ref, ob_ref, fg_ref, fb_ref,
                w1_ref, b1_ref, w2_ref, b2_ref, opw_ref, opb_ref, y_ref):
    h1 = h0_ref[0].astype(F32) + _mmtl(ao_ref[0], ow_ref[...]) + ob_ref[...]
    t = _ln(h1, fg_ref[...], fb_ref[...])
    t = _gelu(_mmtl(t, w1_ref[...]) + b1_ref[...])
    h2 = h1 + _mmtl(t, w2_ref[...]) + b2_ref[...]
    y_ref[0] = _mmtl(h2, opw_ref[...]) + opb_ref[...]


def _full(shape=(D, D)):
    return pl.BlockSpec(shape, lambda *_: tuple(0 for _ in shape))


def kernel(x, state_flat, params):
    p = params
    B, T, _ = x.shape
    state = state_flat.reshape(B, S, D)

    pos = jnp.arange(T, dtype=F32)[:, None]
    f = float(S) ** (jnp.arange(D // 2).astype(F32) / (D // 2))
    pe = jnp.concatenate([jnp.sin(pos / f), jnp.cos(pos / f)], axis=-1)

    r2 = lambda v: v.reshape(1, D)
    smha_w3 = p['smha_w'].reshape(3, D, D)
    omha_w3 = p['omha_w'].reshape(3, D, D)
    bq1, bk1, bv1 = (p['smha_b'][:D][None], p['smha_b'][D:2 * D][None],
                     p['smha_b'][2 * D:][None])
    bq2, bk2, bv2 = (p['omha_b'][:D][None], p['omha_b'][D:2 * D][None],
                     p['omha_b'][2 * D:][None])

    vec = _full((1, D))
    w3 = lambda k, n=1: pl.BlockSpec((n, D, D), lambda *_: (k, 0, 0))
    tile = lambda: pl.BlockSpec((1, TT, D), lambda b, t: (b, t, 0))
    fseq = lambda n: pl.BlockSpec((1, n, D), lambda b: (b, 0, 0))

    k1, v1 = pl.pallas_call(
        _se_proj_body,
        grid=(B, T // TT),
        in_specs=[tile(), pl.BlockSpec((TT, D), lambda b, t: (t, 0)),
                  _full(), vec, vec, vec, w3(1), w3(2), vec, vec],
        out_specs=[tile(), tile()],
        out_shape=[jax.ShapeDtypeStruct((B, T, D), BF16)] * 2,
    )(x, pe, p['se_W'], r2(p['se_b']), r2(p['sln_kv_g']), r2(p['sln_kv_b']),
      smha_w3, smha_w3, bk1, bv1)

    h0, q2, k2, v2 = pl.pallas_call(
        _oe_proj_body,
        grid=(B, T // TT),
        in_specs=[tile(), pl.BlockSpec((TT, D), lambda b, t: (t, 0)),
                  _full(), vec, vec, vec, vec, vec,
                  w3(0, 3), vec, vec, vec],
        out_specs=[tile(), tile(), tile(), tile()],
        out_shape=[jax.ShapeDtypeStruct((B, T, D), BF16)] * 4,
    )(x, pe, p['oe_W'], r2(p['oe_b']), r2(p['oln_q_g']), r2(p['oln_q_b']),
      r2(p['oln_kv_g']), r2(p['oln_kv_b']), omha_w3, bq2, bk2, bv2)

    lr1 = pl.pallas_call(
        _read_attn_body,
        grid=(B,),
        in_specs=[fseq(S), fseq(T), fseq(T), vec, vec, vec, vec, vec,
                  w3(0, 3), vec, vec, vec, _full(), vec],
        out_specs=[fseq(TOPK_READ)],
        out_shape=[jax.ShapeDtypeStruct((B, TOPK_READ, D), F32)],
    )(state, k1, v1, p['read_w'].reshape(1, D),
      r2(p['sln_q_g']), r2(p['sln_q_b']),
      r2(p['sln_kv_g']), r2(p['sln_kv_b']),
      smha_w3, bq1, bk1, bv1, p['smha_ow'], r2(p['smha_ob']))[0]

    sm = lambda n: pl.BlockSpec((1, 1, n), lambda b: (b, 0, 0))
    lr2, eidx, ew, ohe, wmw, wmask = pl.pallas_call(
        _gate_body,
        grid=(B,),
        in_specs=[fseq(TOPK_READ), fseq(S), vec, vec, _full(), vec,
                  _full(), vec, vec, vec, _full((E, D)), vec, vec, vec],
        out_specs=[fseq(TOPK_READ), sm(TOPK), sm(TOPK),
                   pl.BlockSpec((1, TOPK, E), lambda b: (b, 0, 0)),
                   sm(S), sm(S)],
        out_shape=[jax.ShapeDtypeStruct((B, TOPK_READ, D), F32),
                   jax.ShapeDtypeStruct((B, 1, TOPK), jnp.int32),
                   jax.ShapeDtypeStruct((B, 1, TOPK), F32),
                   jax.ShapeDtypeStruct((B, TOPK, E), F32),
                   jax.ShapeDtypeStruct((B, 1, S), F32),
                   jax.ShapeDtypeStruct((B, 1, S), F32)],
    )(lr1, state, r2(p['sln_ffn_g']), r2(p['sln_ffn_b']),
      p['sffn_w1'], r2(p['sffn_b1']), p['sffn_w2'], r2(p['sffn_b2']),
      r2(p['sln_moe_g']), r2(p['sln_moe_b']), p['gate_W'],
      r2(p['sln_slot_g']), r2(p['sln_slot_b']), p['write_w'].reshape(1, D))

    wsel = _gather_experts(p['exp_W'], eidx.reshape(-1))

    st_new, ks, vs = pl.pallas_call(
        _moe_state_body,
        grid=(B,),
        in_specs=[fseq(TOPK_READ),
                  pl.BlockSpec((TOPK, D, D), lambda b: (b, 0, 0)),
                  _full((E, D)),
                  pl.BlockSpec((1, TOPK, E), lambda b: (b, 0, 0)),
                  sm(TOPK), fseq(S), sm(S), sm(S),
                  vec, vec, vec, vec, w3(1), w3(2), vec, vec],
        out_specs=[fseq(S), fseq(S), fseq(S)],
        out_shape=[jax.ShapeDtypeStruct((B, S, D), F32),
                   jax.ShapeDtypeStruct((B, S, D), BF16),
                   jax.ShapeDtypeStruct((B, S, D), BF16)],
    )(lr2, wsel, p['exp_b'], ohe, ew, state, wmw, wmask,
      r2(p['ln_state_g']), r2(p['ln_state_b']),
      r2(p['oln_kv_g']), r2(p['oln_kv_b']), omha_w3, omha_w3, bk2, bv2)

    ao = pl.pallas_call(
        _out_attn_body,
        grid=(B, T // TQ),
        in_specs=[pl.BlockSpec((1, TQ, D), lambda b, t: (b, t, 0)),
                  pl.BlockSpec((1, T, D), lambda b, t: (b, 0, 0)),
                  pl.BlockSpec((1, T, D), lambda b, t: (b, 0, 0)),
                  pl.BlockSpec((1, S, D), lambda b, t: (b, 0, 0)),
                  pl.BlockSpec((1, S, D), lambda b, t: (b, 0, 0))],
        out_specs=[pl.BlockSpec((1, TQ, D), lambda b, t: (b, t, 0))],
        out_shape=[jax.ShapeDtypeStruct((B, T, D), BF16)],
    )(q2, k2, v2, ks, vs)[0]

    y = pl.pallas_call(
        _final_body,
        grid=(B, T // TT),
        in_specs=[tile(), tile(), _full(), vec, vec, vec,
                  _full(), vec, _full(), vec, _full(), vec],
        out_specs=[tile()],
        out_shape=[jax.ShapeDtypeStruct((B, T, D), F32)],
    )(ao, h0, p['omha_ow'], r2(p['omha_ob']),
      r2(p['oln_ffn_g']), r2(p['oln_ffn_b']),
      p['offn_w1'], r2(p['offn_b1']), p['offn_w2'], r2(p['offn_b2']),
      p['op_W'], r2(p['op_b']))

    return y[0], st_new.reshape(B, S * D)
